# baseline (speedup 1.0000x reference)
import functools

import jax
import jax.numpy as jnp
from jax import lax
from jax.experimental import pallas as pl
from jax.experimental.pallas import tpu as pltpu
from jax.experimental.pallas import tpu_sc as plsc

_BATCH = 16384
_EMB = 32
_NC = 2
_NS = 16
_NW = _NC * _NS
_BPW = _BATCH // _NW

_mesh = plsc.VectorSubcoreMesh(core_axis_name="c", subcore_axis_name="s")


@functools.partial(
    pl.kernel,
    mesh=_mesh,
    compiler_params=pltpu.CompilerParams(use_tc_tiling_on_sc=False),
    out_type=[
        jax.ShapeDtypeStruct((_BATCH, _EMB), jnp.float32),
        jax.ShapeDtypeStruct((_BATCH, _EMB), jnp.float32),
    ],
    scratch_types=[
        pltpu.VMEM((_BPW,), jnp.int32),
        pltpu.VMEM((_BPW,), jnp.int32),
        pltpu.VMEM((_BPW, _EMB), jnp.float32),
        pltpu.VMEM((_BPW, _EMB), jnp.float32),
        pltpu.SemaphoreType.DMA,
        pltpu.SemaphoreType.DMA,
    ],
)
def _sc_gather(uid_hbm, mid_hbm, uemb_hbm, memb_hbm, uf_hbm, mf_hbm,
               uidx, midx, urows, mrows, sem_u, sem_m):
    wid = lax.axis_index("s") * _NC + lax.axis_index("c")
    base = wid * _BPW
    pltpu.sync_copy(uid_hbm.at[pl.ds(base, _BPW)], uidx)
    pltpu.sync_copy(mid_hbm.at[pl.ds(base, _BPW)], midx)
    cu = pltpu.async_copy(uemb_hbm.at[uidx], urows, sem_u)
    cm = pltpu.async_copy(memb_hbm.at[midx], mrows, sem_m)
    cu.wait()
    cm.wait()
    pltpu.sync_copy(urows, uf_hbm.at[pl.ds(base, _BPW)])
    pltpu.sync_copy(mrows, mf_hbm.at[pl.ds(base, _BPW)])


_BLK = 2048


def _mlp_body(uf, mf, w1u, w1m, b1, w2, b2, w3, b3, w4, b4, out):
    x = jnp.dot(uf[...], w1u[...], preferred_element_type=jnp.float32)
    x = x + jnp.dot(mf[...], w1m[...], preferred_element_type=jnp.float32)
    x = jnp.maximum(x + b1[...][None, :], 0.0)
    x = jnp.maximum(
        jnp.dot(x, w2[...], preferred_element_type=jnp.float32) + b2[...][None, :], 0.0)
    x = jnp.maximum(
        jnp.dot(x, w3[...], preferred_element_type=jnp.float32) + b3[...][None, :], 0.0)
    y = jnp.dot(x, w4[...], preferred_element_type=jnp.float32)
    out[...] = y[:, 0] + b4[...]


_mlp_call = pl.pallas_call(
    _mlp_body,
    grid=(_BATCH // _BLK,),
    in_specs=[
        pl.BlockSpec((_BLK, _EMB), lambda i: (i, 0)),
        pl.BlockSpec((_BLK, _EMB), lambda i: (i, 0)),
        pl.BlockSpec((_EMB, 32), lambda i: (0, 0)),
        pl.BlockSpec((_EMB, 32), lambda i: (0, 0)),
        pl.BlockSpec((32,), lambda i: (0,)),
        pl.BlockSpec((32, 16), lambda i: (0, 0)),
        pl.BlockSpec((16,), lambda i: (0,)),
        pl.BlockSpec((16, 8), lambda i: (0, 0)),
        pl.BlockSpec((8,), lambda i: (0,)),
        pl.BlockSpec((8, 1), lambda i: (0, 0)),
        pl.BlockSpec((1,), lambda i: (0,)),
    ],
    out_specs=pl.BlockSpec((_BLK,), lambda i: (i,)),
    out_shape=jax.ShapeDtypeStruct((_BATCH,), jnp.float32),
)


def kernel(user_id, movie_id, user_emb, movie_emb, W1, b1, W2, b2, W3, b3, W4, b4):
    uf, mf = _sc_gather(user_id.astype(jnp.int32), movie_id.astype(jnp.int32),
                        user_emb, movie_emb)
    return _mlp_call(uf, mf, W1[:_EMB], W1[_EMB:], b1, W2, b2, W3, b3, W4, b4)
